# Initial kernel scaffold; baseline (speedup 1.0000x reference)
import jax
import jax.numpy as jnp
from jax.experimental import pallas as pl


def kernel(x, edge_vec, edge_attr, edge_length, edge_src, edge_dst, W1, b1, W2, b2, W_sh, W_dir, W_out):
    raise NotImplementedError("write your pallas kernel here")


# speedup vs baseline: 1.3582x; 1.3582x over previous
import functools

import jax
import jax.numpy as jnp
from jax import lax
from jax.experimental import pallas as pl
from jax.experimental.pallas import tpu as pltpu
from jax.experimental.pallas import tpu_sc as plsc

_N = 10000
_E = 320000
_D = 128
_SH = 16
_NRBF = 8
_HID = 64
_RC = 5.0
_AVG_DEG = 32.0

_NC = 2
_NS = 16
_C = 80
_PER_TILE = _E // (_NC * _NS)
_STEPS = _PER_TILE // _C
_RPT = 640
_RPT_LAST = _N - 15 * _RPT
_ZROWS = 80

_TE = 2000
_TN = 2000


def _coeff_body(el_ref, ev_ref, ea_ref, W1_ref, b1_ref, W2_ref, b2_ref,
                Wsh_ref, Wdir_ref, out_ref):
    r = el_ref[...]
    n = lax.broadcasted_iota(jnp.int32, (1, _NRBF), 1).astype(jnp.float32) + 1.0
    rbf = jnp.sqrt(2.0 / _RC) * jnp.sin(n * (jnp.pi / _RC) * r) / (r + 1e-6)
    h = rbf @ W1_ref[...] + b1_ref[...]
    h = h * jax.nn.sigmoid(h)
    radial = h @ W2_ref[...] + b2_ref[...]
    ev = ev_ref[...]
    inv = 1.0 / (jnp.sqrt(jnp.sum(ev * ev, axis=1, keepdims=True)) + 1e-6)
    dirs = ev * inv
    sh_mix = ea_ref[...] @ Wsh_ref[...] + dirs @ Wdir_ref[...]
    out_ref[...] = sh_mix * radial


def _coeff(edge_length, edge_vec, edge_attr, W1, b1, W2, b2, W_sh, W_dir):
    grid = (_E // _TE,)
    full = lambda shape: pl.BlockSpec(shape, lambda i: (0, 0))
    return pl.pallas_call(
        _coeff_body,
        grid=grid,
        in_specs=[
            pl.BlockSpec((_TE, 1), lambda i: (i, 0)),
            pl.BlockSpec((_TE, 3), lambda i: (i, 0)),
            pl.BlockSpec((_TE, _SH), lambda i: (i, 0)),
            full((_NRBF, _HID)),
            full((1, _HID)),
            full((_HID, _D)),
            full((1, _D)),
            full((_SH, _D)),
            full((3, _D)),
        ],
        out_specs=pl.BlockSpec((_TE, _D), lambda i: (i, 0)),
        out_shape=jax.ShapeDtypeStruct((_E, _D), jnp.float32),
    )(edge_length.reshape(_E, 1), edge_vec, edge_attr,
      W1, b1.reshape(1, _HID), W2, b2.reshape(1, _D), W_sh, W_dir)


def _sc_body(x_hbm, coeff_hbm, src_hbm, dst_hbm, out_hbm,
             sidx, didx, hj, cf, zb, acc):
    c = lax.axis_index("c")
    s = lax.axis_index("s")

    @pl.loop(0, _ZROWS)
    def _zero(r):
        for j in range(_D // 16):
            zb[r, pl.ds(j * 16, 16)] = jnp.zeros((16,), jnp.float32)

    for k in range(_RPT_LAST // _ZROWS):
        pltpu.sync_copy(zb, acc.at[pl.ds(s * _RPT + k * _ZROWS, _ZROWS)])

    @pl.when(s < 15)
    def _zero_rest():
        for k in range(_RPT_LAST // _ZROWS, _RPT // _ZROWS):
            pltpu.sync_copy(zb, acc.at[pl.ds(s * _RPT + k * _ZROWS, _ZROWS)])

    plsc.subcore_barrier()

    base_row = (c * _NS + s) * _STEPS

    @pl.loop(0, _STEPS)
    def _step(i):
        row = base_row + i
        pltpu.sync_copy(src_hbm.at[row], sidx.at[0])
        pltpu.sync_copy(dst_hbm.at[row], didx.at[0])
        pltpu.sync_copy(x_hbm.at[sidx.at[0]], hj)
        pltpu.sync_copy(coeff_hbm.at[pl.ds(row * _C, _C)], cf)

        @pl.loop(0, _C)
        def _mul(r):
            for j in range(_D // 16):
                sl = pl.ds(j * 16, 16)
                hj[r, sl] = hj[r, sl] * cf[r, sl]

        pltpu.sync_copy(hj, acc.at[didx.at[0]], add=True)

    plsc.subcore_barrier()

    @pl.when(s < 15)
    def _dump():
        pltpu.sync_copy(acc.at[pl.ds(s * _RPT, _RPT)],
                        out_hbm.at[c, pl.ds(s * _RPT, _RPT)])

    @pl.when(s == 15)
    def _dump_last():
        pltpu.sync_copy(acc.at[pl.ds(15 * _RPT, _RPT_LAST)],
                        out_hbm.at[c, pl.ds(15 * _RPT, _RPT_LAST)])


def _sc_aggregate(x, coeff, src2d, dst2d):
    mesh = plsc.VectorSubcoreMesh(core_axis_name="c", subcore_axis_name="s")
    kern = pl.kernel(
        _sc_body,
        out_type=jax.ShapeDtypeStruct((_NC, _N, _D), jnp.float32),
        mesh=mesh,
        scratch_types=[
            pltpu.VMEM((1, _C), jnp.int32),
            pltpu.VMEM((1, _C), jnp.int32),
            pltpu.VMEM((_C, _D), jnp.float32),
            pltpu.VMEM((_C, _D), jnp.float32),
            pltpu.VMEM((_ZROWS, _D), jnp.float32),
            pltpu.VMEM_SHARED((_N, _D), jnp.float32),
        ],
    )
    return kern(x, coeff, src2d, dst2d)


def _final_body(p_ref, x_ref, Wout_ref, o_ref):
    agg = (p_ref[0] + p_ref[1]) * (1.0 / jnp.sqrt(_AVG_DEG))
    o_ref[...] = agg @ Wout_ref[...] + x_ref[...]


def _final(partial, x, W_out):
    grid = (_N // _TN,)
    return pl.pallas_call(
        _final_body,
        grid=grid,
        in_specs=[
            pl.BlockSpec((_NC, _TN, _D), lambda i: (0, i, 0)),
            pl.BlockSpec((_TN, _D), lambda i: (i, 0)),
            pl.BlockSpec((_D, _D), lambda i: (0, 0)),
        ],
        out_specs=pl.BlockSpec((_TN, _D), lambda i: (i, 0)),
        out_shape=jax.ShapeDtypeStruct((_N, _D), jnp.float32),
    )(partial, x, W_out)


def kernel(x, edge_vec, edge_attr, edge_length, edge_src, edge_dst,
           W1, b1, W2, b2, W_sh, W_dir, W_out):
    coeff = _coeff(edge_length, edge_vec, edge_attr, W1, b1, W2, b2,
                   W_sh, W_dir)
    src2d = edge_src.astype(jnp.int32).reshape(_E // _C, _C)
    dst2d = edge_dst.astype(jnp.int32).reshape(_E // _C, _C)
    partial = _sc_aggregate(x, coeff, src2d, dst2d)
    return _final(partial, x, W_out)


# speedup vs baseline: 2.4639x; 1.8140x over previous
import functools

import jax
import jax.numpy as jnp
from jax import lax
from jax.experimental import pallas as pl
from jax.experimental.pallas import tpu as pltpu
from jax.experimental.pallas import tpu_sc as plsc

_N = 10000
_E = 320000
_D = 128
_SH = 16
_NRBF = 8
_HID = 64
_RC = 5.0
_AVG_DEG = 32.0

_NC = 2
_NS = 16
_C = 80
_PER_TILE = _E // (_NC * _NS)
_STEPS = _PER_TILE // _C
_RPT = 640
_RPT_LAST = _N - 15 * _RPT
_ZROWS = 80

_TE = 2560
_TN = 2000


def _coeff_body(el_ref, evT_ref, ea_ref, W1_ref, b1_ref, W2_ref, b2_ref,
                Wsh_ref, Wdir_ref, out_ref):
    r = el_ref[...]
    u = jnp.sqrt(2.0 / _RC) / (r + 1e-6)
    n = lax.broadcasted_iota(jnp.int32, (_NRBF, 1), 0).astype(jnp.float32) + 1.0
    thetaT = n * ((jnp.pi / _RC) * r)
    rbfT = jnp.sin(thetaT) * u
    h = lax.dot_general(rbfT, W1_ref[...],
                        (((0,), (0,)), ((), ()))) + b1_ref[...]
    h = h * jax.nn.sigmoid(h)
    radial = h @ W2_ref[...] + b2_ref[...]
    evT = evT_ref[...]
    inv = 1.0 / (jnp.sqrt(jnp.sum(evT * evT, axis=0, keepdims=True)) + 1e-6)
    dirsT = evT * inv
    sh_mix = (ea_ref[...] @ Wsh_ref[...]
              + lax.dot_general(dirsT, Wdir_ref[...],
                                (((0,), (0,)), ((), ()))))
    out_ref[...] = sh_mix * radial


def _coeff(edge_length, edge_vec, edge_attr, W1, b1, W2, b2, W_sh, W_dir):
    grid = (_E // _TE,)
    full = lambda shape: pl.BlockSpec(shape, lambda i: (0, 0))
    return pl.pallas_call(
        _coeff_body,
        grid=grid,
        in_specs=[
            pl.BlockSpec((1, _TE), lambda i: (0, i)),
            pl.BlockSpec((3, _TE), lambda i: (0, i)),
            pl.BlockSpec((_TE, _SH), lambda i: (i, 0)),
            full((_NRBF, _HID)),
            full((1, _HID)),
            full((_HID, _D)),
            full((1, _D)),
            full((_SH, _D)),
            full((3, _D)),
        ],
        out_specs=pl.BlockSpec((_TE, _D), lambda i: (i, 0)),
        out_shape=jax.ShapeDtypeStruct((_E, _D), jnp.float32),
    )(edge_length.reshape(1, _E), edge_vec.T, edge_attr,
      W1, b1.reshape(1, _HID), W2, b2.reshape(1, _D), W_sh, W_dir)


def _sc_body(x_hbm, coeff_hbm, src_hbm, dst_hbm, out_hbm,
             sidx, didx, hj, cf, zb, acc):
    c = lax.axis_index("c")
    s = lax.axis_index("s")

    @pl.loop(0, _ZROWS)
    def _zero(r):
        for j in range(_D // 16):
            zb[r, pl.ds(j * 16, 16)] = jnp.zeros((16,), jnp.float32)

    for k in range(_RPT_LAST // _ZROWS):
        pltpu.sync_copy(zb, acc.at[pl.ds(s * _RPT + k * _ZROWS, _ZROWS)])

    @pl.when(s < 15)
    def _zero_rest():
        for k in range(_RPT_LAST // _ZROWS, _RPT // _ZROWS):
            pltpu.sync_copy(zb, acc.at[pl.ds(s * _RPT + k * _ZROWS, _ZROWS)])

    plsc.subcore_barrier()

    base_row = (c * _NS + s) * _STEPS

    @pl.loop(0, _STEPS)
    def _step(i):
        row = base_row + i
        pltpu.sync_copy(src_hbm.at[row], sidx.at[0])
        pltpu.sync_copy(dst_hbm.at[row], didx.at[0])
        pltpu.sync_copy(x_hbm.at[sidx.at[0]], hj)
        pltpu.sync_copy(coeff_hbm.at[pl.ds(row * _C, _C)], cf)

        @pl.loop(0, _C)
        def _mul(r):
            for j in range(_D // 16):
                sl = pl.ds(j * 16, 16)
                hj[r, sl] = hj[r, sl] * cf[r, sl]

        pltpu.sync_copy(hj, acc.at[didx.at[0]], add=True)

    plsc.subcore_barrier()

    @pl.when(s < 15)
    def _dump():
        pltpu.sync_copy(acc.at[pl.ds(s * _RPT, _RPT)],
                        out_hbm.at[c, pl.ds(s * _RPT, _RPT)])

    @pl.when(s == 15)
    def _dump_last():
        pltpu.sync_copy(acc.at[pl.ds(15 * _RPT, _RPT_LAST)],
                        out_hbm.at[c, pl.ds(15 * _RPT, _RPT_LAST)])


def _sc_aggregate(x, coeff, src2d, dst2d):
    mesh = plsc.VectorSubcoreMesh(core_axis_name="c", subcore_axis_name="s")
    kern = pl.kernel(
        _sc_body,
        out_type=jax.ShapeDtypeStruct((_NC, _N, _D), jnp.float32),
        mesh=mesh,
        scratch_types=[
            pltpu.VMEM((1, _C), jnp.int32),
            pltpu.VMEM((1, _C), jnp.int32),
            pltpu.VMEM((_C, _D), jnp.float32),
            pltpu.VMEM((_C, _D), jnp.float32),
            pltpu.VMEM((_ZROWS, _D), jnp.float32),
            pltpu.VMEM_SHARED((_N, _D), jnp.float32),
        ],
    )
    return kern(x, coeff, src2d, dst2d)


def _final_body(p_ref, x_ref, Wout_ref, o_ref):
    agg = (p_ref[0] + p_ref[1]) * (1.0 / jnp.sqrt(_AVG_DEG))
    o_ref[...] = agg @ Wout_ref[...] + x_ref[...]


def _final(partial, x, W_out):
    grid = (_N // _TN,)
    return pl.pallas_call(
        _final_body,
        grid=grid,
        in_specs=[
            pl.BlockSpec((_NC, _TN, _D), lambda i: (0, i, 0)),
            pl.BlockSpec((_TN, _D), lambda i: (i, 0)),
            pl.BlockSpec((_D, _D), lambda i: (0, 0)),
        ],
        out_specs=pl.BlockSpec((_TN, _D), lambda i: (i, 0)),
        out_shape=jax.ShapeDtypeStruct((_N, _D), jnp.float32),
    )(partial, x, W_out)


def kernel(x, edge_vec, edge_attr, edge_length, edge_src, edge_dst,
           W1, b1, W2, b2, W_sh, W_dir, W_out):
    coeff = _coeff(edge_length, edge_vec, edge_attr, W1, b1, W2, b2,
                   W_sh, W_dir)
    src2d = edge_src.astype(jnp.int32).reshape(_E // _C, _C)
    dst2d = edge_dst.astype(jnp.int32).reshape(_E // _C, _C)
    partial = _sc_aggregate(x, coeff, src2d, dst2d)
    return _final(partial, x, W_out)


# speedup vs baseline: 2.4726x; 1.0035x over previous
import functools

import jax
import jax.numpy as jnp
from jax import lax
from jax.experimental import pallas as pl
from jax.experimental.pallas import tpu as pltpu
from jax.experimental.pallas import tpu_sc as plsc

_N = 10000
_E = 320000
_D = 128
_SH = 16
_NRBF = 8
_HID = 64
_RC = 5.0
_AVG_DEG = 32.0

_NC = 2
_NS = 16
_NW = _NC * _NS
_C = 64
_STEPS = 160
_EP = _NW * _STEPS * _C
_ACC_N = _N + 16
_RPT = 640
_RPT_LAST = _ACC_N - 15 * _RPT
_ZROWS = 8

_TE = 2560
_TN = 2000


def _coeff_body(el_ref, evT_ref, ea_ref, W1_ref, b1_ref, W2_ref, b2_ref,
                Wsh_ref, Wdir_ref, out_ref):
    r = el_ref[0].reshape(1, _TE)
    u = jnp.sqrt(2.0 / _RC) / (r + 1e-6)
    n = lax.broadcasted_iota(jnp.int32, (_NRBF, 1), 0).astype(jnp.float32) + 1.0
    thetaT = n * ((jnp.pi / _RC) * r)
    rbfT = jnp.sin(thetaT) * u
    h = lax.dot_general(rbfT, W1_ref[...],
                        (((0,), (0,)), ((), ()))) + b1_ref[...]
    h = h * jax.nn.sigmoid(h)
    radial = h @ W2_ref[...] + b2_ref[...]
    evT = evT_ref[...]
    inv = 1.0 / (jnp.sqrt(jnp.sum(evT * evT, axis=0, keepdims=True)) + 1e-6)
    dirsT = evT * inv
    sh_mix = (ea_ref[...] @ Wsh_ref[...]
              + lax.dot_general(dirsT, Wdir_ref[...],
                                (((0,), (0,)), ((), ()))))
    out_ref[...] = sh_mix * radial


def _coeff(edge_length, edge_vec, edge_attr, W1, b1, W2, b2, W_sh, W_dir):
    grid = (_EP // _TE,)
    nreal = _E // _TE - 1
    clamp = lambda i: jnp.minimum(i, nreal)
    full = lambda shape: pl.BlockSpec(shape, lambda i: (0, 0))
    return pl.pallas_call(
        _coeff_body,
        grid=grid,
        in_specs=[
            pl.BlockSpec((1, _TE // 128, 128), lambda i: (clamp(i), 0, 0)),
            pl.BlockSpec((3, _TE), lambda i: (0, clamp(i))),
            pl.BlockSpec((_TE, _SH), lambda i: (clamp(i), 0)),
            full((_NRBF, _HID)),
            full((1, _HID)),
            full((_HID, _D)),
            full((1, _D)),
            full((_SH, _D)),
            full((3, _D)),
        ],
        out_specs=pl.BlockSpec((_TE, _D), lambda i: (i, 0)),
        out_shape=jax.ShapeDtypeStruct((_EP, _D), jnp.float32),
    )(edge_length.reshape(_E // _TE, _TE // 128, 128), edge_vec.T, edge_attr,
      W1, b1.reshape(1, _HID), W2, b2.reshape(1, _D), W_sh, W_dir)


def _sc_body(x_hbm, coeff_hbm, src_hbm, dst_hbm, out_hbm,
             si0, si1, si2, si3, di0, di1, di2, di3,
             hj0, hj1, cf0, cf1, ms0, ms1, acc,
             i0, i1, i2, i3, g0, g1, s0, s1):
    c = lax.axis_index("c")
    s = lax.axis_index("s")
    wid = c * _NS + s
    sis, dis = (si0, si1, si2, si3), (di0, di1, di2, di3)
    hjs, cfs, msgs = (hj0, hj1), (cf0, cf1), (ms0, ms1)
    isem, gsem, ssem = (i0, i1, i2, i3), (g0, g1), (s0, s1)

    @pl.loop(0, _C)
    def _zero(r):
        for j in range(_D // 16):
            hj0[r, pl.ds(j * 16, 16)] = jnp.zeros((16,), jnp.float32)

    for k in range(_RPT // _C):
        @pl.when((s < 15) if k >= _RPT_LAST // _C else (s < 16))
        def _z():
            pltpu.sync_copy(hj0, acc.at[pl.ds(s * _RPT + k * _C, _C)])

    @pl.when(s == 15)
    def _zero_tail():
        pltpu.sync_copy(hj0.at[pl.ds(0, 32)],
                        acc.at[pl.ds(15 * _RPT + 6 * _C, 32)])

    plsc.subcore_barrier()

    base = wid * _STEPS

    def _fetch_idx(i, q):
        off = (base + i) * _C
        pltpu.async_copy(src_hbm.at[pl.ds(off, _C)], sis[q].at[0], isem[q])
        pltpu.async_copy(dst_hbm.at[pl.ds(off, _C)], dis[q].at[0], isem[q])

    def _fetch(i, b, q):
        pltpu.make_async_copy(src_hbm.at[pl.ds(0, _C)], sis[q].at[0],
                              isem[q]).wait()
        pltpu.make_async_copy(src_hbm.at[pl.ds(0, _C)], dis[q].at[0],
                              isem[q]).wait()
        pltpu.async_copy(x_hbm.at[sis[q].at[0]], hjs[b], gsem[b])
        pltpu.async_copy(coeff_hbm.at[pl.ds((base + i) * _C, _C)],
                         cfs[b], gsem[b])

    def _process(i, b, q, prefetch, wait_msg=True):
        pltpu.make_async_copy(x_hbm.at[pl.ds(0, _C)], hjs[b], gsem[b]).wait()
        pltpu.make_async_copy(x_hbm.at[pl.ds(0, _C)], cfs[b], gsem[b]).wait()
        if wait_msg:
            pltpu.make_async_copy(x_hbm.at[pl.ds(0, _C)], msgs[b],
                                  ssem[b]).wait()
        if prefetch is not None:
            _fetch_idx(prefetch, (q + 2) % 4)

        @pl.loop(0, _C)
        def _mul(r):
            for j in range(_D // 16):
                sl = pl.ds(j * 16, 16)
                msgs[b][r, sl] = hjs[b][r, sl] * cfs[b][r, sl]

        pltpu.async_copy(msgs[b], acc.at[dis[q].at[0]], ssem[b], add=True)
        if prefetch is not None:
            _fetch(prefetch, b, (q + 2) % 4)

    _fetch_idx(0, 0)
    _fetch_idx(1, 1)
    _fetch(0, 0, 0)
    _fetch(1, 1, 1)
    _process(0, 0, 0, 2, wait_msg=False)
    _process(1, 1, 1, 3, wait_msg=False)

    @pl.loop(0, (_STEPS - 8) // 4)
    def _main(m):
        ib = 4 * m + 2
        for j in range(4):
            _process(ib + j, j % 2, (2 + j) % 4, ib + j + 2)

    for j in range(4):
        i = _STEPS - 6 + j
        _process(i, i % 2, i % 4, i + 2)
    _process(_STEPS - 2, 0, (_STEPS - 2) % 4, None)
    _process(_STEPS - 1, 1, (_STEPS - 1) % 4, None)

    pltpu.make_async_copy(x_hbm.at[pl.ds(0, _C)], ms0, s0).wait()
    pltpu.make_async_copy(x_hbm.at[pl.ds(0, _C)], ms1, s1).wait()

    plsc.subcore_barrier()

    @pl.when(s < 15)
    def _dump():
        pltpu.sync_copy(acc.at[pl.ds(s * _RPT, _RPT)],
                        out_hbm.at[c, pl.ds(s * _RPT, _RPT)])

    @pl.when(s == 15)
    def _dump_last():
        pltpu.sync_copy(acc.at[pl.ds(15 * _RPT, _N - 15 * _RPT)],
                        out_hbm.at[c, pl.ds(15 * _RPT, _N - 15 * _RPT)])


def _sc_aggregate(x, coeff, src1d, dst1d):
    mesh = plsc.VectorSubcoreMesh(core_axis_name="c", subcore_axis_name="s")
    kern = pl.kernel(
        _sc_body,
        out_type=jax.ShapeDtypeStruct((_NC, _N, _D), jnp.float32),
        mesh=mesh,
        scratch_types=(
            [pltpu.VMEM((1, _C), jnp.int32) for _ in range(8)]
            + [pltpu.VMEM((_C, _D), jnp.float32) for _ in range(6)]
            + [pltpu.VMEM_SHARED((_ACC_N, _D), jnp.float32)]
            + [pltpu.SemaphoreType.DMA for _ in range(8)]
        ),
    )
    return kern(x, coeff, src1d, dst1d)


def _final_body(p_ref, x_ref, Wout_ref, o_ref):
    agg = (p_ref[0] + p_ref[1]) * (1.0 / jnp.sqrt(_AVG_DEG))
    o_ref[...] = agg @ Wout_ref[...] + x_ref[...]


def _final(partial, x, W_out):
    grid = (_N // _TN,)
    return pl.pallas_call(
        _final_body,
        grid=grid,
        in_specs=[
            pl.BlockSpec((_NC, _TN, _D), lambda i: (0, i, 0)),
            pl.BlockSpec((_TN, _D), lambda i: (i, 0)),
            pl.BlockSpec((_D, _D), lambda i: (0, 0)),
        ],
        out_specs=pl.BlockSpec((_TN, _D), lambda i: (i, 0)),
        out_shape=jax.ShapeDtypeStruct((_N, _D), jnp.float32),
    )(partial, x, W_out)


def kernel(x, edge_vec, edge_attr, edge_length, edge_src, edge_dst,
           W1, b1, W2, b2, W_sh, W_dir, W_out):
    coeff = _coeff(edge_length, edge_vec, edge_attr, W1, b1, W2, b2,
                   W_sh, W_dir)
    src1d = jnp.pad(edge_src.astype(jnp.int32), (0, _EP - _E))
    dst1d = jnp.pad(edge_dst.astype(jnp.int32), (0, _EP - _E),
                    constant_values=_N)
    partial = _sc_aggregate(x, coeff, src1d, dst1d)
    return _final(partial, x, W_out)
